# Initial kernel scaffold; baseline (speedup 1.0000x reference)
import jax
import jax.numpy as jnp
from jax import lax
from jax.experimental import pallas as pl
from jax.experimental.pallas import tpu as pltpu
from jax.experimental.pallas import tpu_sc as plsc

N_ROWS = 128
N = 32768
L = 16
NV = N // L
NB = 512
NBV = NB // L
NC = 2
NS = 16
NW = NC * NS
ROWS_PER_W = N_ROWS // NW

COARSE_W = 1.0 / NB
FINE_W = 1.0 / (NB * NB)


def _zero_tables(cnt_v, sum_v):
    def body(i, _):
        z = jnp.zeros((L,), jnp.float32)
        cnt_v[pl.ds(i * L, L)] = z
        sum_v[pl.ds(i * L, L)] = z
        return 0
    lax.fori_loop(0, NBV, body, 0)


def _suffix_scan(cnt_v, sum_v, base, width, carry_k, carry_s):
    def body(i, carry):
        ck, cs, npos = carry
        j = NBV - 1 - i
        kv = cnt_v[pl.ds(j * L, L)]
        sv = sum_v[pl.ds(j * L, L)]
        ksuf = jnp.flip(jnp.cumsum(jnp.flip(kv))) + ck
        ssuf = jnp.flip(jnp.cumsum(jnp.flip(sv))) + cs
        cnt_v[pl.ds(j * L, L)] = ksuf
        sum_v[pl.ds(j * L, L)] = ssuf
        idx = j * L + lax.iota(jnp.int32, L)
        edge = base + idx.astype(jnp.float32) * width
        f = ssuf - ksuf * edge - 1.0
        npos = npos + jnp.sum(jnp.where(f > 0.0, 1.0, 0.0))
        return (ck + jnp.sum(kv), cs + jnp.sum(sv), npos)

    _, _, npos = lax.fori_loop(
        0, NBV, body, (carry_k, carry_s, jnp.float32(0.0)))
    jstar = jnp.maximum(npos.astype(jnp.int32) - 1, 0)
    jv = jnp.full((L,), jstar, jnp.int32)
    k_at = jnp.max(plsc.load_gather(cnt_v, [jv]))
    s_at = jnp.max(plsc.load_gather(sum_v, [jv]))
    jv1 = jnp.minimum(jv + 1, NB - 1)
    top = jstar >= NB - 1
    k_up = jnp.where(top, 0.0, jnp.max(plsc.load_gather(cnt_v, [jv1])))
    s_up = jnp.where(top, 0.0, jnp.max(plsc.load_gather(sum_v, [jv1])))
    return jstar, k_at, s_at, k_up, s_up


def _sparsemax_body(x_hbm, out_hbm, row_v, cnt_v, sum_v):
    wid = lax.axis_index("s") * NC + lax.axis_index("c")

    for r in range(ROWS_PER_W):
        row = wid * ROWS_PER_W + r
        pltpu.sync_copy(x_hbm.at[row], row_v)

        def max_body(i, m):
            return jnp.maximum(m, row_v[pl.ds(i * L, L)])
        mv = lax.fori_loop(0, NV, max_body,
                           jnp.full((L,), -jnp.inf, jnp.float32))
        lo = jnp.max(mv) - 1.0

        _zero_tables(cnt_v, sum_v)

        def coarse_body(i, _):
            xv = row_v[pl.ds(i * L, L)]
            t = (xv - lo) * jnp.float32(NB)
            b = jnp.minimum(jnp.maximum(t.astype(jnp.int32), 0), NB - 1)
            msk = xv > lo
            plsc.addupdate_scatter(cnt_v, [b], jnp.ones((L,), jnp.float32),
                                   mask=msk)
            plsc.addupdate_scatter(sum_v, [b], xv, mask=msk)
            return 0
        lax.fori_loop(0, NV, coarse_body, 0)

        jstar, _, _, k_up, s_up = _suffix_scan(
            cnt_v, sum_v, lo, jnp.float32(COARSE_W),
            jnp.float32(0.0), jnp.float32(0.0))

        flo = lo + jstar.astype(jnp.float32) * jnp.float32(COARSE_W)
        _zero_tables(cnt_v, sum_v)

        def fine_body(i, _):
            xv = row_v[pl.ds(i * L, L)]
            bc = jnp.minimum(jnp.maximum(
                ((xv - lo) * jnp.float32(NB)).astype(jnp.int32), 0), NB - 1)
            msk = (xv > lo) & (bc == jstar)
            t = (xv - flo) * jnp.float32(NB * NB)
            b = jnp.minimum(jnp.maximum(t.astype(jnp.int32), 0), NB - 1)
            plsc.addupdate_scatter(cnt_v, [b], jnp.ones((L,), jnp.float32),
                                   mask=msk)
            plsc.addupdate_scatter(sum_v, [b], xv, mask=msk)
            return 0
        lax.fori_loop(0, NV, fine_body, 0)

        _, k_f, s_f, _, _ = _suffix_scan(
            cnt_v, sum_v, flo, jnp.float32(FINE_W), k_up, s_up)

        tau = (s_f - 1.0) / jnp.maximum(k_f, 1.0)

        def out_body(i, _):
            xv = row_v[pl.ds(i * L, L)]
            row_v[pl.ds(i * L, L)] = jnp.maximum(xv - tau, 0.0)
            return 0
        lax.fori_loop(0, NV, out_body, 0)
        pltpu.sync_copy(row_v, out_hbm.at[row])


def kernel(x):
    mesh = plsc.VectorSubcoreMesh(core_axis_name="c", subcore_axis_name="s")
    run = pl.kernel(
        _sparsemax_body,
        mesh=mesh,
        out_type=jax.ShapeDtypeStruct((N_ROWS, N), jnp.float32),
        scratch_types=[
            pltpu.VMEM((N,), jnp.float32),
            pltpu.VMEM((NB,), jnp.float32),
            pltpu.VMEM((NB,), jnp.float32),
        ],
    )
    return run(x)


# speedup vs baseline: 6.3005x; 6.3005x over previous
import jax
import jax.numpy as jnp
from jax import lax
from jax.experimental import pallas as pl
from jax.experimental.pallas import tpu as pltpu
from jax.experimental.pallas import tpu_sc as plsc

N_ROWS = 128
N = 32768
L = 16
NV = N // L
NB = 512
NBV = NB // L
NC = 2
NS = 16
NW = NC * NS
ROWS_PER_W = N_ROWS // NW

COARSE_W = 1.0 / NB
FINE_W = 1.0 / (NB * NB)


def _zero_tables(cnt_v, sum_v):
    def body(i, _):
        z = jnp.zeros((L,), jnp.float32)
        cnt_v[pl.ds(i * L, L)] = z
        sum_v[pl.ds(i * L, L)] = z
        return 0
    lax.fori_loop(0, NBV, body, 0)


def _suffix_scan(cnt_v, sum_v, base, width, carry_k, carry_s):
    def body(i, carry):
        ck, cs, npos = carry
        j = NBV - 1 - i
        kv = cnt_v[pl.ds(j * L, L)]
        sv = sum_v[pl.ds(j * L, L)]
        ksuf = jnp.flip(jnp.cumsum(jnp.flip(kv))) + ck
        ssuf = jnp.flip(jnp.cumsum(jnp.flip(sv))) + cs
        cnt_v[pl.ds(j * L, L)] = ksuf
        sum_v[pl.ds(j * L, L)] = ssuf
        idx = j * L + lax.iota(jnp.int32, L)
        edge = base + idx.astype(jnp.float32) * width
        f = ssuf - ksuf * edge - 1.0
        npos = npos + jnp.sum(jnp.where(f > 0.0, 1.0, 0.0))
        return (ck + jnp.sum(kv), cs + jnp.sum(sv), npos)

    _, _, npos = lax.fori_loop(
        0, NBV, body, (carry_k, carry_s, jnp.float32(0.0)))
    jstar = jnp.maximum(npos.astype(jnp.int32) - 1, 0)
    jv = jnp.full((L,), jstar, jnp.int32)
    k_at = jnp.max(plsc.load_gather(cnt_v, [jv]))
    s_at = jnp.max(plsc.load_gather(sum_v, [jv]))
    jv1 = jnp.minimum(jv + 1, NB - 1)
    top = jstar >= NB - 1
    k_up = jnp.where(top, 0.0, jnp.max(plsc.load_gather(cnt_v, [jv1])))
    s_up = jnp.where(top, 0.0, jnp.max(plsc.load_gather(sum_v, [jv1])))
    return jstar, k_at, s_at, k_up, s_up


def _sparsemax_body(x_hbm, out_hbm, row_v, cnt_v, sum_v):
    wid = lax.axis_index("s") * NC + lax.axis_index("c")

    for r in range(ROWS_PER_W):
        row = wid * ROWS_PER_W + r
        pltpu.sync_copy(x_hbm.at[row], row_v)

        def max_body(i, m):
            return jnp.maximum(m, row_v[pl.ds(i * L, L)])
        mv = lax.fori_loop(0, NV, max_body,
                           jnp.full((L,), -jnp.inf, jnp.float32))
        lo = jnp.max(mv) - 1.0

        _zero_tables(cnt_v, sum_v)

        def coarse_body(i, _):
            xv = row_v[pl.ds(i * L, L)]
            t = (xv - lo) * jnp.float32(NB)
            b = jnp.minimum(jnp.maximum(t.astype(jnp.int32), 0), NB - 1)
            msk = xv > lo
            plsc.addupdate_scatter(cnt_v, [b], jnp.ones((L,), jnp.float32),
                                   mask=msk)
            plsc.addupdate_scatter(sum_v, [b], xv, mask=msk)
            return 0
        lax.fori_loop(0, NV, coarse_body, 0)

        jstar, _, _, k_up, s_up = _suffix_scan(
            cnt_v, sum_v, lo, jnp.float32(COARSE_W),
            jnp.float32(0.0), jnp.float32(0.0))

        flo = lo + jstar.astype(jnp.float32) * jnp.float32(COARSE_W)
        _zero_tables(cnt_v, sum_v)

        def fine_body(i, _):
            xv = row_v[pl.ds(i * L, L)]
            bc = jnp.minimum(jnp.maximum(
                ((xv - lo) * jnp.float32(NB)).astype(jnp.int32), 0), NB - 1)
            msk = (xv > lo) & (bc == jstar)
            t = (xv - flo) * jnp.float32(NB * NB)
            b = jnp.minimum(jnp.maximum(t.astype(jnp.int32), 0), NB - 1)
            plsc.addupdate_scatter(cnt_v, [b], jnp.ones((L,), jnp.float32),
                                   mask=msk)
            plsc.addupdate_scatter(sum_v, [b], xv, mask=msk)
            return 0
        lax.fori_loop(0, NV, fine_body, 0)

        _, k_f, s_f, _, _ = _suffix_scan(
            cnt_v, sum_v, flo, jnp.float32(FINE_W), k_up, s_up)

        num_v = jnp.full((L,), s_f - 1.0, jnp.float32)
        den_v = jnp.maximum(jnp.full((L,), k_f, jnp.float32), 1.0)
        tau = jnp.max(num_v / den_v)

        def out_body(i, _):
            xv = row_v[pl.ds(i * L, L)]
            row_v[pl.ds(i * L, L)] = jnp.maximum(xv - tau, 0.0)
            return 0
        lax.fori_loop(0, NV, out_body, 0)
        pltpu.sync_copy(row_v, out_hbm.at[row])


def kernel(x):
    mesh = plsc.VectorSubcoreMesh(core_axis_name="c", subcore_axis_name="s")
    run = pl.kernel(
        _sparsemax_body,
        mesh=mesh,
        compiler_params=pltpu.CompilerParams(needs_layout_passes=False),
        out_type=jax.ShapeDtypeStruct((N_ROWS, N), jnp.float32),
        scratch_types=[
            pltpu.VMEM((N,), jnp.float32),
            pltpu.VMEM((NB,), jnp.float32),
            pltpu.VMEM((NB,), jnp.float32),
        ],
    )
    return run(x)


# speedup vs baseline: 32.0291x; 5.0835x over previous
import jax
import jax.numpy as jnp
from jax import lax
from jax.experimental import pallas as pl
from jax.experimental.pallas import tpu as pltpu
from jax.experimental.pallas import tpu_sc as plsc

N_ROWS = 128
N = 32768
L = 16
NV = N // L
NB = 512
NBV = NB // L
NC = 2
NS = 16
NW = NC * NS
ROWS_PER_W = N_ROWS // NW
CAP = 4096

COARSE_W = 1.0 / NB
FINE_W = 1.0 / (NB * NB)


def _suffix_scan(cnt_v, sum_v, base, width, carry_k, carry_s):
    def body(i, carry):
        ck, cs, npos = carry
        j = NBV - 1 - i
        kv = cnt_v[pl.ds(j * L, L)]
        sv = sum_v[pl.ds(j * L, L)]
        ksuf = jnp.flip(jnp.cumsum(jnp.flip(kv))) + ck
        ssuf = jnp.flip(jnp.cumsum(jnp.flip(sv))) + cs
        cnt_v[pl.ds(j * L, L)] = ksuf
        sum_v[pl.ds(j * L, L)] = ssuf
        idx = j * L + lax.iota(jnp.int32, L)
        edge = base + idx.astype(jnp.float32) * width
        f = ssuf - ksuf * edge - 1.0
        npos = npos + plsc.all_reduce_population_count(f > 0.0)
        return (ck + jnp.sum(kv), cs + jnp.sum(sv), npos)

    _, _, npos = lax.fori_loop(
        0, NBV, body,
        (carry_k, carry_s, jnp.zeros((L,), jnp.int32)))
    jstar = jnp.maximum(jnp.max(npos) - 1, 0)
    jv = jnp.full((L,), jstar, jnp.int32)
    k_at = jnp.max(plsc.load_gather(cnt_v, [jv]))
    s_at = jnp.max(plsc.load_gather(sum_v, [jv]))
    jv1 = jnp.minimum(jv + 1, NB - 1)
    top = jstar >= NB - 1
    k_up = jnp.where(top, 0.0, jnp.max(plsc.load_gather(cnt_v, [jv1])))
    s_up = jnp.where(top, 0.0, jnp.max(plsc.load_gather(sum_v, [jv1])))
    return jstar, k_at, s_at, k_up, s_up


def _zero_tables(cnt_v, sum_v):
    def body(i):
        z = jnp.zeros((L,), jnp.float32)
        cnt_v[pl.ds(i * L, L)] = z
        sum_v[pl.ds(i * L, L)] = z
    plsc.parallel_loop(0, NBV)(body)


def _row_tau(buf, cmp_v, cnt_v, sum_v):
    def max_body(i, ms):
        return tuple(jnp.maximum(ms[j], buf[pl.ds((i + j) * L, L)])
                     for j in range(8))
    ms = plsc.parallel_loop(
        0, NV, 8, unroll=2,
        carry=tuple(jnp.full((L,), -jnp.inf, jnp.float32) for _ in range(8))
    )(max_body)
    m01 = jnp.maximum(jnp.maximum(ms[0], ms[1]), jnp.maximum(ms[2], ms[3]))
    m23 = jnp.maximum(jnp.maximum(ms[4], ms[5]), jnp.maximum(ms[6], ms[7]))
    lo = jnp.max(jnp.maximum(m01, m23)) - 1.0

    def comp_body(i, cb):
        xv = buf[pl.ds(i * L, L)]
        msk = xv > lo
        mf = jnp.where(msk, jnp.float32(1.0), jnp.float32(0.0))
        pos = plsc.cumsum(mf).astype(jnp.int32)
        idx = jnp.minimum(jnp.maximum(cb + pos - 1, 0), CAP - 1)
        plsc.store_scatter(cmp_v, [idx], xv, mask=msk)
        return cb + plsc.all_reduce_population_count(msk)
    cb = plsc.parallel_loop(
        0, NV, unroll=4, carry=jnp.zeros((L,), jnp.int32))(comp_body)
    nc = jnp.minimum(jnp.max(cb), CAP)
    tc = (nc + (L - 1)) >> 4

    lanes = lax.iota(jnp.int32, L)
    ones = jnp.ones((L,), jnp.float32)

    _zero_tables(cnt_v, sum_v)

    def chist_body(i, _):
        xv = cmp_v[pl.ds(i * L, L)]
        valid = (i * L + lanes) < nc
        t = (xv - lo) * jnp.float32(NB)
        b = jnp.minimum(jnp.maximum(t.astype(jnp.int32), 0), NB - 1)
        plsc.addupdate_scatter(cnt_v, [b], ones, mask=valid)
        plsc.addupdate_scatter(sum_v, [b], xv, mask=valid)
        return 0
    lax.fori_loop(0, tc, chist_body, 0)

    jstar, _, _, k_up, s_up = _suffix_scan(
        cnt_v, sum_v, lo, jnp.float32(COARSE_W),
        jnp.float32(0.0), jnp.float32(0.0))

    flo = lo + jstar.astype(jnp.float32) * jnp.float32(COARSE_W)
    _zero_tables(cnt_v, sum_v)

    def fhist_body(i, _):
        xv = cmp_v[pl.ds(i * L, L)]
        bc = jnp.minimum(jnp.maximum(
            ((xv - lo) * jnp.float32(NB)).astype(jnp.int32), 0), NB - 1)
        msk = ((i * L + lanes) < nc) & (bc == jstar)
        t = (xv - flo) * jnp.float32(NB * NB)
        b = jnp.minimum(jnp.maximum(t.astype(jnp.int32), 0), NB - 1)
        plsc.addupdate_scatter(cnt_v, [b], ones, mask=msk)
        plsc.addupdate_scatter(sum_v, [b], xv, mask=msk)
        return 0
    lax.fori_loop(0, tc, fhist_body, 0)

    _, k_f, s_f, _, _ = _suffix_scan(
        cnt_v, sum_v, flo, jnp.float32(FINE_W), k_up, s_up)

    num_v = jnp.full((L,), s_f - 1.0, jnp.float32)
    den_v = jnp.maximum(jnp.full((L,), k_f, jnp.float32), 1.0)
    return jnp.max(num_v / den_v)


def _sparsemax_body(x_hbm, out_hbm, buf0, buf1, buf2, cmp_v, cnt_v, sum_v,
                    in_sems, out_sems):
    bufs = (buf0, buf1, buf2)
    wid = lax.axis_index("s") * NC + lax.axis_index("c")
    base_row = wid * ROWS_PER_W

    in_h = {0: pltpu.async_copy(x_hbm.at[base_row], bufs[0], in_sems.at[0])}
    out_h = {}
    for r in range(ROWS_PER_W):
        buf = bufs[r % 3]
        if r + 1 < ROWS_PER_W:
            nxt = (r + 1) % 3
            if r + 1 >= 3:
                out_h[r - 2].wait()
            in_h[r + 1] = pltpu.async_copy(
                x_hbm.at[base_row + r + 1], bufs[nxt], in_sems.at[nxt])
        in_h[r].wait()

        tau = _row_tau(buf, cmp_v, cnt_v, sum_v)

        def out_body(i):
            for j in range(8):
                xv = buf[pl.ds((i + j) * L, L)]
                buf[pl.ds((i + j) * L, L)] = jnp.maximum(xv - tau, 0.0)
        plsc.parallel_loop(0, NV, 8, unroll=2)(out_body)

        out_h[r] = pltpu.async_copy(
            buf, out_hbm.at[base_row + r], out_sems.at[r % 3])
    for r in range(max(ROWS_PER_W - 3, 1), ROWS_PER_W):
        out_h[r].wait()


def kernel(x):
    mesh = plsc.VectorSubcoreMesh(core_axis_name="c", subcore_axis_name="s")
    run = pl.kernel(
        _sparsemax_body,
        mesh=mesh,
        compiler_params=pltpu.CompilerParams(needs_layout_passes=False),
        out_type=jax.ShapeDtypeStruct((N_ROWS, N), jnp.float32),
        scratch_types=[
            pltpu.VMEM((N,), jnp.float32),
            pltpu.VMEM((N,), jnp.float32),
            pltpu.VMEM((N,), jnp.float32),
            pltpu.VMEM((CAP,), jnp.float32),
            pltpu.VMEM((NB,), jnp.float32),
            pltpu.VMEM((NB,), jnp.float32),
            pltpu.SemaphoreType.DMA((3,)),
            pltpu.SemaphoreType.DMA((3,)),
        ],
    )
    return run(x)


# speedup vs baseline: 32.9428x; 1.0285x over previous
import jax
import jax.numpy as jnp
from jax import lax
from jax.experimental import pallas as pl
from jax.experimental.pallas import tpu as pltpu
from jax.experimental.pallas import tpu_sc as plsc

N_ROWS = 128
N = 32768
L = 16
NV = N // L
NBL = 64
NBLV = NBL // L
NLVL = 3
NC = 2
NS = 16
NW = NC * NS
ROWS_PER_W = N_ROWS // NW
CAP = 4096


def _suffix_scan(cnt_v, sum_v, base, width, carry_k, carry_s):
    def body(i, carry):
        ck, cs, npos = carry
        j = NBLV - 1 - i
        kv = cnt_v[pl.ds(j * L, L)]
        sv = sum_v[pl.ds(j * L, L)]
        ksuf = jnp.flip(jnp.cumsum(jnp.flip(kv))) + ck
        ssuf = jnp.flip(jnp.cumsum(jnp.flip(sv))) + cs
        cnt_v[pl.ds(j * L, L)] = ksuf
        sum_v[pl.ds(j * L, L)] = ssuf
        idx = j * L + lax.iota(jnp.int32, L)
        edge = base + idx.astype(jnp.float32) * width
        f = ssuf - ksuf * edge - 1.0
        npos = npos + plsc.all_reduce_population_count(f > 0.0)
        return (ck + jnp.sum(kv), cs + jnp.sum(sv), npos)

    _, _, npos = lax.fori_loop(
        0, NBLV, body,
        (carry_k, carry_s, jnp.zeros((L,), jnp.int32)))
    jstar = jnp.maximum(jnp.max(npos) - 1, 0)
    jv = jnp.full((L,), jstar, jnp.int32)
    k_at = jnp.max(plsc.load_gather(cnt_v, [jv]))
    s_at = jnp.max(plsc.load_gather(sum_v, [jv]))
    jv1 = jnp.minimum(jv + 1, NBL - 1)
    top = jstar >= NBL - 1
    k_up = jnp.where(top, carry_k, jnp.max(plsc.load_gather(cnt_v, [jv1])))
    s_up = jnp.where(top, carry_s, jnp.max(plsc.load_gather(sum_v, [jv1])))
    return jstar, k_at, s_at, k_up, s_up


def _row_tau(buf, cmp_v, cnt_v, sum_v):
    lanes = lax.iota(jnp.int32, L)
    ones = jnp.ones((L,), jnp.float32)

    def max_body(i, ms):
        return tuple(jnp.maximum(ms[j], buf[pl.ds((i + j) * L, L)])
                     for j in range(8))
    ms = plsc.parallel_loop(
        0, NV, 8, unroll=2,
        carry=tuple(jnp.full((L,), -jnp.inf, jnp.float32) for _ in range(8))
    )(max_body)
    m01 = jnp.maximum(jnp.maximum(ms[0], ms[1]), jnp.maximum(ms[2], ms[3]))
    m23 = jnp.maximum(jnp.maximum(ms[4], ms[5]), jnp.maximum(ms[6], ms[7]))
    lo = jnp.max(jnp.maximum(m01, m23)) - 1.0

    def comp_body(i, cb):
        xv = buf[pl.ds(i * L, L)]
        msk = xv > lo
        mf = jnp.where(msk, jnp.float32(1.0), jnp.float32(0.0))
        pos = plsc.cumsum(mf).astype(jnp.int32)
        idx = jnp.minimum(jnp.maximum(cb + pos - 1, 0), CAP - 1)
        plsc.store_scatter(cmp_v, [idx], xv, mask=msk)
        return cb + plsc.all_reduce_population_count(msk)
    cb = plsc.parallel_loop(
        0, NV, unroll=4, carry=jnp.zeros((L,), jnp.int32))(comp_body)
    nc = jnp.minimum(jnp.max(cb), CAP)
    tc = (nc + (L - 1)) >> 4

    base = lo
    scale = 1.0
    prev = []
    k_up = jnp.float32(0.0)
    s_up = jnp.float32(0.0)
    k_at = jnp.float32(0.0)
    s_at = jnp.float32(0.0)
    for _ in range(NLVL):
        scale = scale * NBL
        width = 1.0 / scale

        def zero_body(i):
            z = jnp.zeros((L,), jnp.float32)
            cnt_v[pl.ds(i * L, L)] = z
            sum_v[pl.ds(i * L, L)] = z
        plsc.parallel_loop(0, NBLV)(zero_body)

        def hist_body(i, _, base=base, scale=scale, prev=tuple(prev)):
            xv = cmp_v[pl.ds(i * L, L)]
            msk = ((i * L + lanes) < nc) & (xv > lo)
            for (pb, ps, pj) in prev:
                pbin = jnp.minimum(jnp.maximum(
                    ((xv - pb) * jnp.float32(ps)).astype(jnp.int32), 0),
                    NBL - 1)
                msk = msk & (pbin == pj)
            b = jnp.minimum(jnp.maximum(
                ((xv - base) * jnp.float32(scale)).astype(jnp.int32), 0),
                NBL - 1)
            plsc.addupdate_scatter(cnt_v, [b], ones, mask=msk)
            plsc.addupdate_scatter(sum_v, [b], xv, mask=msk)
            return 0
        lax.fori_loop(0, tc, hist_body, 0)

        jstar, k_at, s_at, k_up, s_up = _suffix_scan(
            cnt_v, sum_v, base, jnp.float32(width), k_up, s_up)
        prev.append((base, scale, jstar))
        base = base + jstar.astype(jnp.float32) * jnp.float32(width)

    num_v = jnp.full((L,), s_at - 1.0, jnp.float32)
    den_v = jnp.maximum(jnp.full((L,), k_at, jnp.float32), 1.0)
    return jnp.max(num_v / den_v)


def _sparsemax_body(x_hbm, out_hbm, buf0, buf1, buf2, cmp_v, cnt_v, sum_v,
                    in_sems, out_sems):
    bufs = (buf0, buf1, buf2)
    wid = lax.axis_index("s") * NC + lax.axis_index("c")
    base_row = wid * ROWS_PER_W

    in_h = {0: pltpu.async_copy(x_hbm.at[base_row], bufs[0], in_sems.at[0])}
    out_h = {}
    for r in range(ROWS_PER_W):
        buf = bufs[r % 3]
        if r + 1 < ROWS_PER_W:
            nxt = (r + 1) % 3
            if r + 1 >= 3:
                out_h[r - 2].wait()
            in_h[r + 1] = pltpu.async_copy(
                x_hbm.at[base_row + r + 1], bufs[nxt], in_sems.at[nxt])
        in_h[r].wait()

        tau = _row_tau(buf, cmp_v, cnt_v, sum_v)

        def out_body(i):
            for j in range(8):
                xv = buf[pl.ds((i + j) * L, L)]
                buf[pl.ds((i + j) * L, L)] = jnp.maximum(xv - tau, 0.0)
        plsc.parallel_loop(0, NV, 8, unroll=2)(out_body)

        out_h[r] = pltpu.async_copy(
            buf, out_hbm.at[base_row + r], out_sems.at[r % 3])
    for r in range(max(ROWS_PER_W - 3, 1), ROWS_PER_W):
        out_h[r].wait()


def kernel(x):
    mesh = plsc.VectorSubcoreMesh(core_axis_name="c", subcore_axis_name="s")
    run = pl.kernel(
        _sparsemax_body,
        mesh=mesh,
        compiler_params=pltpu.CompilerParams(needs_layout_passes=False),
        out_type=jax.ShapeDtypeStruct((N_ROWS, N), jnp.float32),
        scratch_types=[
            pltpu.VMEM((N,), jnp.float32),
            pltpu.VMEM((N,), jnp.float32),
            pltpu.VMEM((N,), jnp.float32),
            pltpu.VMEM((CAP,), jnp.float32),
            pltpu.VMEM((NBL,), jnp.float32),
            pltpu.VMEM((NBL,), jnp.float32),
            pltpu.SemaphoreType.DMA((3,)),
            pltpu.SemaphoreType.DMA((3,)),
        ],
    )
    return run(x)


# speedup vs baseline: 34.3100x; 1.0415x over previous
import jax
import jax.numpy as jnp
from jax import lax
from jax.experimental import pallas as pl
from jax.experimental.pallas import tpu as pltpu
from jax.experimental.pallas import tpu_sc as plsc

N_ROWS = 128
N = 32768
L = 16
NV = N // L
NBL = 64
NBLV = NBL // L
NLVL = 3
NC = 2
NS = 16
NW = NC * NS
ROWS_PER_W = N_ROWS // NW
CAP = 4096


def _suffix_scan(cnt_v, sum_v, base, width, carry_k, carry_s):
    def body(i, carry):
        ck, cs, npos = carry
        j = NBLV - 1 - i
        kv = cnt_v[pl.ds(j * L, L)]
        sv = sum_v[pl.ds(j * L, L)]
        ksuf = jnp.flip(jnp.cumsum(jnp.flip(kv))) + ck
        ssuf = jnp.flip(jnp.cumsum(jnp.flip(sv))) + cs
        cnt_v[pl.ds(j * L, L)] = ksuf
        sum_v[pl.ds(j * L, L)] = ssuf
        idx = j * L + lax.iota(jnp.int32, L)
        edge = base + idx.astype(jnp.float32) * width
        f = ssuf - ksuf * edge - 1.0
        npos = npos + plsc.all_reduce_population_count(f > 0.0)
        return (ck + jnp.sum(kv), cs + jnp.sum(sv), npos)

    _, _, npos = lax.fori_loop(
        0, NBLV, body,
        (carry_k, carry_s, jnp.zeros((L,), jnp.int32)))
    jstar = jnp.maximum(jnp.max(npos) - 1, 0)
    jv = jnp.full((L,), jstar, jnp.int32)
    k_at = jnp.max(plsc.load_gather(cnt_v, [jv]))
    s_at = jnp.max(plsc.load_gather(sum_v, [jv]))
    jv1 = jnp.minimum(jv + 1, NBL - 1)
    top = jstar >= NBL - 1
    k_up = jnp.where(top, carry_k, jnp.max(plsc.load_gather(cnt_v, [jv1])))
    s_up = jnp.where(top, carry_s, jnp.max(plsc.load_gather(sum_v, [jv1])))
    return jstar, k_at, s_at, k_up, s_up


def _row_tau(buf, cmp_v, cnt_v, sum_v):
    lanes = lax.iota(jnp.int32, L)
    ones = jnp.ones((L,), jnp.float32)

    def max_body(i, ms):
        return tuple(jnp.maximum(ms[j], buf[pl.ds((i + j) * L, L)])
                     for j in range(8))
    ms = plsc.parallel_loop(
        0, NV, 8, unroll=2,
        carry=tuple(jnp.full((L,), -jnp.inf, jnp.float32) for _ in range(8))
    )(max_body)
    m01 = jnp.maximum(jnp.maximum(ms[0], ms[1]), jnp.maximum(ms[2], ms[3]))
    m23 = jnp.maximum(jnp.maximum(ms[4], ms[5]), jnp.maximum(ms[6], ms[7]))
    lo = jnp.max(jnp.maximum(m01, m23)) - 1.0

    def comp_body(i, cb):
        xv = buf[pl.ds(i * L, L)]
        p = plsc.all_reduce_population_count(xv > lo)
        any_v = p > 0
        idx = jnp.minimum(cb + lanes, CAP - 1)
        plsc.store_scatter(cmp_v, [idx], xv, mask=any_v)
        return cb + jnp.where(any_v, L, 0)
    cb = plsc.parallel_loop(
        0, NV, unroll=4, carry=jnp.zeros((L,), jnp.int32))(comp_body)
    nc = jnp.minimum(jnp.max(cb), CAP)
    tc = (nc + (L - 1)) >> 4

    base = lo
    scale = 1.0
    prev = []
    k_up = jnp.float32(0.0)
    s_up = jnp.float32(0.0)
    k_at = jnp.float32(0.0)
    s_at = jnp.float32(0.0)
    for _ in range(NLVL):
        scale = scale * NBL
        width = 1.0 / scale

        def zero_body(i):
            z = jnp.zeros((L,), jnp.float32)
            cnt_v[pl.ds(i * L, L)] = z
            sum_v[pl.ds(i * L, L)] = z
        plsc.parallel_loop(0, NBLV)(zero_body)

        def hist_body(i, _, base=base, scale=scale, prev=tuple(prev)):
            xv = cmp_v[pl.ds(i * L, L)]
            msk = ((i * L + lanes) < nc) & (xv > lo)
            for (pb, ps, pj) in prev:
                pbin = jnp.minimum(jnp.maximum(
                    ((xv - pb) * jnp.float32(ps)).astype(jnp.int32), 0),
                    NBL - 1)
                msk = msk & (pbin == pj)
            b = jnp.minimum(jnp.maximum(
                ((xv - base) * jnp.float32(scale)).astype(jnp.int32), 0),
                NBL - 1)
            plsc.addupdate_scatter(cnt_v, [b], ones, mask=msk)
            plsc.addupdate_scatter(sum_v, [b], xv, mask=msk)
            return 0
        lax.fori_loop(0, tc, hist_body, 0)

        jstar, k_at, s_at, k_up, s_up = _suffix_scan(
            cnt_v, sum_v, base, jnp.float32(width), k_up, s_up)
        prev.append((base, scale, jstar))
        base = base + jstar.astype(jnp.float32) * jnp.float32(width)

    num_v = jnp.full((L,), s_at - 1.0, jnp.float32)
    den_v = jnp.maximum(jnp.full((L,), k_at, jnp.float32), 1.0)
    return jnp.max(num_v / den_v)


def _sparsemax_body(x_hbm, out_hbm, buf0, buf1, buf2, cmp_v, cnt_v, sum_v,
                    in_sems, out_sems):
    bufs = (buf0, buf1, buf2)
    wid = lax.axis_index("s") * NC + lax.axis_index("c")
    base_row = wid * ROWS_PER_W

    in_h = {0: pltpu.async_copy(x_hbm.at[base_row], bufs[0], in_sems.at[0])}
    out_h = {}
    for r in range(ROWS_PER_W):
        buf = bufs[r % 3]
        if r + 1 < ROWS_PER_W:
            nxt = (r + 1) % 3
            if r + 1 >= 3:
                out_h[r - 2].wait()
            in_h[r + 1] = pltpu.async_copy(
                x_hbm.at[base_row + r + 1], bufs[nxt], in_sems.at[nxt])
        in_h[r].wait()

        tau = _row_tau(buf, cmp_v, cnt_v, sum_v)

        def out_body(i):
            for j in range(8):
                xv = buf[pl.ds((i + j) * L, L)]
                buf[pl.ds((i + j) * L, L)] = jnp.maximum(xv - tau, 0.0)
        plsc.parallel_loop(0, NV, 8, unroll=2)(out_body)

        out_h[r] = pltpu.async_copy(
            buf, out_hbm.at[base_row + r], out_sems.at[r % 3])
    for r in range(max(ROWS_PER_W - 3, 1), ROWS_PER_W):
        out_h[r].wait()


def kernel(x):
    mesh = plsc.VectorSubcoreMesh(core_axis_name="c", subcore_axis_name="s")
    run = pl.kernel(
        _sparsemax_body,
        mesh=mesh,
        compiler_params=pltpu.CompilerParams(needs_layout_passes=False),
        out_type=jax.ShapeDtypeStruct((N_ROWS, N), jnp.float32),
        scratch_types=[
            pltpu.VMEM((N,), jnp.float32),
            pltpu.VMEM((N,), jnp.float32),
            pltpu.VMEM((N,), jnp.float32),
            pltpu.VMEM((CAP,), jnp.float32),
            pltpu.VMEM((NBL,), jnp.float32),
            pltpu.VMEM((NBL,), jnp.float32),
            pltpu.SemaphoreType.DMA((3,)),
            pltpu.SemaphoreType.DMA((3,)),
        ],
    )
    return run(x)


# speedup vs baseline: 36.8798x; 1.0749x over previous
import jax
import jax.numpy as jnp
from jax import lax
from jax.experimental import pallas as pl
from jax.experimental.pallas import tpu as pltpu
from jax.experimental.pallas import tpu_sc as plsc

N_ROWS = 128
N = 32768
L = 16
NV = N // L
NBL = 64
NBLV = NBL // L
NLVL = 3
NC = 2
NS = 16
NW = NC * NS
ROWS_PER_W = N_ROWS // NW


def _suffix_scan(cnt_v, sum_v, base, width, carry_k, carry_s):
    def body(i, carry):
        ck, cs, npos = carry
        j = NBLV - 1 - i
        kv = cnt_v[pl.ds(j * L, L)]
        sv = sum_v[pl.ds(j * L, L)]
        ksuf = jnp.flip(jnp.cumsum(jnp.flip(kv))) + ck
        ssuf = jnp.flip(jnp.cumsum(jnp.flip(sv))) + cs
        cnt_v[pl.ds(j * L, L)] = ksuf
        sum_v[pl.ds(j * L, L)] = ssuf
        idx = j * L + lax.iota(jnp.int32, L)
        edge = base + idx.astype(jnp.float32) * width
        f = ssuf - ksuf * edge - 1.0
        npos = npos + plsc.all_reduce_population_count(f > 0.0)
        return (ck + jnp.sum(kv), cs + jnp.sum(sv), npos)

    _, _, npos = lax.fori_loop(
        0, NBLV, body,
        (carry_k, carry_s, jnp.zeros((L,), jnp.int32)))
    jstar = jnp.maximum(jnp.max(npos) - 1, 0)
    jv = jnp.full((L,), jstar, jnp.int32)
    k_at = jnp.max(plsc.load_gather(cnt_v, [jv]))
    s_at = jnp.max(plsc.load_gather(sum_v, [jv]))
    jv1 = jnp.minimum(jv + 1, NBL - 1)
    top = jstar >= NBL - 1
    k_up = jnp.where(top, carry_k, jnp.max(plsc.load_gather(cnt_v, [jv1])))
    s_up = jnp.where(top, carry_s, jnp.max(plsc.load_gather(sum_v, [jv1])))
    return jstar, k_at, s_at, k_up, s_up


def _row_max(buf):
    def max_body(i, ms):
        return tuple(jnp.maximum(ms[j], buf[pl.ds((i + j) * L, L)])
                     for j in range(8))
    ms = plsc.parallel_loop(
        0, NV, 8, unroll=2,
        carry=tuple(jnp.full((L,), -jnp.inf, jnp.float32) for _ in range(8))
    )(max_body)
    m01 = jnp.maximum(jnp.maximum(ms[0], ms[1]), jnp.maximum(ms[2], ms[3]))
    m23 = jnp.maximum(jnp.maximum(ms[4], ms[5]), jnp.maximum(ms[6], ms[7]))
    return jnp.max(jnp.maximum(m01, m23))


def _row_tau(buf, lo, cmp_v, cnt_v, sum_v):
    lanes = lax.iota(jnp.int32, L)
    ones = jnp.ones((L,), jnp.float32)

    def comp_body(i, cb):
        xv = buf[pl.ds(i * L, L)]
        p = plsc.all_reduce_population_count(xv > lo)
        any_v = p > 0
        plsc.store_scatter(cmp_v, [cb + lanes], xv, mask=any_v)
        return cb + jnp.where(any_v, L, 0)
    cb = plsc.parallel_loop(
        0, NV, unroll=8, carry=jnp.zeros((L,), jnp.int32))(comp_body)
    nc = jnp.max(cb)
    tc = (nc + (L - 1)) >> 4

    base = lo
    scale = 1.0
    prev = []
    k_up = jnp.float32(0.0)
    s_up = jnp.float32(0.0)
    k_at = jnp.float32(0.0)
    s_at = jnp.float32(0.0)
    for _ in range(NLVL):
        scale = scale * NBL
        width = 1.0 / scale

        def zero_body(i):
            z = jnp.zeros((L,), jnp.float32)
            cnt_v[pl.ds(i * L, L)] = z
            sum_v[pl.ds(i * L, L)] = z
        plsc.parallel_loop(0, NBLV)(zero_body)

        def hist_body(i, _, base=base, scale=scale, prev=tuple(prev)):
            xv = cmp_v[pl.ds(i * L, L)]
            msk = ((i * L + lanes) < nc) & (xv > lo)
            for (pb, ps, pj) in prev:
                pbin = jnp.minimum(jnp.maximum(
                    ((xv - pb) * jnp.float32(ps)).astype(jnp.int32), 0),
                    NBL - 1)
                msk = msk & (pbin == pj)
            b = jnp.minimum(jnp.maximum(
                ((xv - base) * jnp.float32(scale)).astype(jnp.int32), 0),
                NBL - 1)
            plsc.addupdate_scatter(cnt_v, [b], ones, mask=msk)
            plsc.addupdate_scatter(sum_v, [b], xv, mask=msk)
            return 0
        lax.fori_loop(0, tc, hist_body, 0)

        jstar, k_at, s_at, k_up, s_up = _suffix_scan(
            cnt_v, sum_v, base, jnp.float32(width), k_up, s_up)
        prev.append((base, scale, jstar))
        base = base + jstar.astype(jnp.float32) * jnp.float32(width)

    num_v = jnp.full((L,), s_at - 1.0, jnp.float32)
    den_v = jnp.maximum(jnp.full((L,), k_at, jnp.float32), 1.0)
    return jnp.max(num_v / den_v)


def _sparsemax_body(x_hbm, out_hbm, buf0, buf1, cmp_v, cnt_v, sum_v,
                    in_sems, out_sems):
    bufs = (buf0, buf1)
    wid = lax.axis_index("s") * NC + lax.axis_index("c")
    base_row = wid * ROWS_PER_W

    in_h = {0: pltpu.async_copy(x_hbm.at[base_row], bufs[0], in_sems.at[0])}
    out_h = {}
    for r in range(ROWS_PER_W):
        buf = bufs[r % 2]
        in_h[r].wait()
        lo = _row_max(buf) - 1.0
        if r + 1 < ROWS_PER_W:
            nxt = (r + 1) % 2
            if r >= 1:
                out_h[r - 1].wait()
            in_h[r + 1] = pltpu.async_copy(
                x_hbm.at[base_row + r + 1], bufs[nxt], in_sems.at[nxt])

        tau = _row_tau(buf, lo, cmp_v, cnt_v, sum_v)

        def out_body(i):
            for j in range(8):
                xv = buf[pl.ds((i + j) * L, L)]
                buf[pl.ds((i + j) * L, L)] = jnp.maximum(xv - tau, 0.0)
        plsc.parallel_loop(0, NV, 8, unroll=2)(out_body)

        out_h[r] = pltpu.async_copy(
            buf, out_hbm.at[base_row + r], out_sems.at[r % 2])
    out_h[ROWS_PER_W - 2].wait()
    out_h[ROWS_PER_W - 1].wait()


def kernel(x):
    mesh = plsc.VectorSubcoreMesh(core_axis_name="c", subcore_axis_name="s")
    run = pl.kernel(
        _sparsemax_body,
        mesh=mesh,
        compiler_params=pltpu.CompilerParams(needs_layout_passes=False),
        out_type=jax.ShapeDtypeStruct((N_ROWS, N), jnp.float32),
        scratch_types=[
            pltpu.VMEM((N,), jnp.float32),
            pltpu.VMEM((N,), jnp.float32),
            pltpu.VMEM((N,), jnp.float32),
            pltpu.VMEM((NBL,), jnp.float32),
            pltpu.VMEM((NBL,), jnp.float32),
            pltpu.SemaphoreType.DMA((2,)),
            pltpu.SemaphoreType.DMA((2,)),
        ],
    )
    return run(x)


# speedup vs baseline: 37.2023x; 1.0087x over previous
import jax
import jax.numpy as jnp
from jax import lax
from jax.experimental import pallas as pl
from jax.experimental.pallas import tpu as pltpu
from jax.experimental.pallas import tpu_sc as plsc

N_ROWS = 128
N = 32768
L = 16
NV = N // L
NBL = 64
NBLV = NBL // L
NLVL = 3
NC = 2
NS = 16
NW = NC * NS
ROWS_PER_W = N_ROWS // NW


def _suffix_scan(cnt_v, sum_v, base, width, carry_k, carry_s):
    def body(i, carry):
        ck, cs, npos = carry
        j = NBLV - 1 - i
        kv = cnt_v[pl.ds(j * L, L)]
        sv = sum_v[pl.ds(j * L, L)]
        ksuf = jnp.flip(jnp.cumsum(jnp.flip(kv))) + ck
        ssuf = jnp.flip(jnp.cumsum(jnp.flip(sv))) + cs
        cnt_v[pl.ds(j * L, L)] = ksuf
        sum_v[pl.ds(j * L, L)] = ssuf
        idx = j * L + lax.iota(jnp.int32, L)
        edge = base + idx.astype(jnp.float32) * width
        f = ssuf - ksuf * edge - 1.0
        npos = npos + plsc.all_reduce_population_count(f > 0.0)
        return (ck + jnp.sum(kv), cs + jnp.sum(sv), npos)

    carry = (carry_k, carry_s, jnp.zeros((L,), jnp.int32))
    for i in range(NBLV):
        carry = body(i, carry)
    _, _, npos = carry
    jstar = jnp.maximum(jnp.max(npos) - 1, 0)
    jv = jnp.full((L,), jstar, jnp.int32)
    k_at = jnp.max(plsc.load_gather(cnt_v, [jv]))
    s_at = jnp.max(plsc.load_gather(sum_v, [jv]))
    jv1 = jnp.minimum(jv + 1, NBL - 1)
    top = jstar >= NBL - 1
    k_up = jnp.where(top, carry_k, jnp.max(plsc.load_gather(cnt_v, [jv1])))
    s_up = jnp.where(top, carry_s, jnp.max(plsc.load_gather(sum_v, [jv1])))
    return jstar, k_at, s_at, k_up, s_up


def _row_max(buf):
    def max_body(i, ms):
        return tuple(jnp.maximum(ms[j], buf[pl.ds((i + j) * L, L)])
                     for j in range(8))
    ms = plsc.parallel_loop(
        0, NV, 8, unroll=2,
        carry=tuple(jnp.full((L,), -jnp.inf, jnp.float32) for _ in range(8))
    )(max_body)
    m01 = jnp.maximum(jnp.maximum(ms[0], ms[1]), jnp.maximum(ms[2], ms[3]))
    m23 = jnp.maximum(jnp.maximum(ms[4], ms[5]), jnp.maximum(ms[6], ms[7]))
    return jnp.max(jnp.maximum(m01, m23))


def _row_tau(buf, lo, cmp_v, cnt_v, sum_v):
    lanes = lax.iota(jnp.int32, L)
    ones = jnp.ones((L,), jnp.float32)

    def comp_body(i, cb):
        xv = buf[pl.ds(i * L, L)]
        p = plsc.all_reduce_population_count(xv > lo)
        any_v = p > 0
        plsc.store_scatter(cmp_v, [cb + lanes], xv, mask=any_v)
        return cb + jnp.where(any_v, L, 0)
    cb = plsc.parallel_loop(
        0, NV, unroll=8, carry=jnp.zeros((L,), jnp.int32))(comp_body)
    nc = jnp.max(cb)
    tc = (nc + (L - 1)) >> 4

    base = lo
    scale = 1.0
    prev = []
    k_up = jnp.float32(0.0)
    s_up = jnp.float32(0.0)
    k_at = jnp.float32(0.0)
    s_at = jnp.float32(0.0)
    for _ in range(NLVL):
        scale = scale * NBL
        width = 1.0 / scale

        for i in range(NBLV):
            z = jnp.zeros((L,), jnp.float32)
            cnt_v[pl.ds(i * L, L)] = z
            sum_v[pl.ds(i * L, L)] = z

        def hist_body(i, _, base=base, scale=scale, prev=tuple(prev)):
            for j in range(4):
                iv = i * 4 + j
                xv = cmp_v[pl.ds(iv * L, L)]
                msk = ((iv * L + lanes) < nc) & (xv > lo)
                for (pb, ps, pj) in prev:
                    pbin = jnp.minimum(jnp.maximum(
                        ((xv - pb) * jnp.float32(ps)).astype(jnp.int32), 0),
                        NBL - 1)
                    msk = msk & (pbin == pj)
                b = jnp.minimum(jnp.maximum(
                    ((xv - base) * jnp.float32(scale)).astype(jnp.int32), 0),
                    NBL - 1)
                plsc.addupdate_scatter(cnt_v, [b], ones, mask=msk)
                plsc.addupdate_scatter(sum_v, [b], xv, mask=msk)
            return 0
        lax.fori_loop(0, (tc + 3) >> 2, hist_body, 0)

        jstar, k_at, s_at, k_up, s_up = _suffix_scan(
            cnt_v, sum_v, base, jnp.float32(width), k_up, s_up)
        prev.append((base, scale, jstar))
        base = base + jstar.astype(jnp.float32) * jnp.float32(width)

    num_v = jnp.full((L,), s_at - 1.0, jnp.float32)
    den_v = jnp.maximum(jnp.full((L,), k_at, jnp.float32), 1.0)
    return jnp.max(num_v / den_v)


def _sparsemax_body(x_hbm, out_hbm, buf0, buf1, cmp_v, cnt_v, sum_v,
                    in_sems, out_sems):
    bufs = (buf0, buf1)
    wid = lax.axis_index("s") * NC + lax.axis_index("c")
    base_row = wid * ROWS_PER_W

    in_h = {0: pltpu.async_copy(x_hbm.at[base_row], bufs[0], in_sems.at[0])}
    out_h = {}
    for r in range(ROWS_PER_W):
        buf = bufs[r % 2]
        in_h[r].wait()
        lo = _row_max(buf) - 1.0
        if r + 1 < ROWS_PER_W:
            nxt = (r + 1) % 2
            if r >= 1:
                out_h[r - 1].wait()
            in_h[r + 1] = pltpu.async_copy(
                x_hbm.at[base_row + r + 1], bufs[nxt], in_sems.at[nxt])

        tau = _row_tau(buf, lo, cmp_v, cnt_v, sum_v)

        def out_body(i):
            for j in range(8):
                xv = buf[pl.ds((i + j) * L, L)]
                buf[pl.ds((i + j) * L, L)] = jnp.maximum(xv - tau, 0.0)
        plsc.parallel_loop(0, NV, 8, unroll=2)(out_body)

        out_h[r] = pltpu.async_copy(
            buf, out_hbm.at[base_row + r], out_sems.at[r % 2])
    out_h[ROWS_PER_W - 2].wait()
    out_h[ROWS_PER_W - 1].wait()


def kernel(x):
    mesh = plsc.VectorSubcoreMesh(core_axis_name="c", subcore_axis_name="s")
    run = pl.kernel(
        _sparsemax_body,
        mesh=mesh,
        compiler_params=pltpu.CompilerParams(needs_layout_passes=False),
        out_type=jax.ShapeDtypeStruct((N_ROWS, N), jnp.float32),
        scratch_types=[
            pltpu.VMEM((N,), jnp.float32),
            pltpu.VMEM((N,), jnp.float32),
            pltpu.VMEM((N + 64,), jnp.float32),
            pltpu.VMEM((NBL,), jnp.float32),
            pltpu.VMEM((NBL,), jnp.float32),
            pltpu.SemaphoreType.DMA((2,)),
            pltpu.SemaphoreType.DMA((2,)),
        ],
    )
    return run(x)


# speedup vs baseline: 40.7653x; 1.0958x over previous
import jax
import jax.numpy as jnp
from jax import lax
from jax.experimental import pallas as pl
from jax.experimental.pallas import tpu as pltpu
from jax.experimental.pallas import tpu_sc as plsc

N_ROWS = 128
N = 32768
L = 16
NV = N // L
NBL = 64
NBLV = NBL // L
NLVL = 3
NC = 2
NS = 16
NW = NC * NS
ROWS_PER_W = N_ROWS // NW
CAP2 = 8192


def _suffix_scan(cnt_v, sum_v, base, width, carry_k, carry_s):
    def body(i, carry):
        ck, cs, npos = carry
        j = NBLV - 1 - i
        kv = cnt_v[pl.ds(j * L, L)]
        sv = sum_v[pl.ds(j * L, L)]
        ksuf = jnp.flip(jnp.cumsum(jnp.flip(kv))) + ck
        ssuf = jnp.flip(jnp.cumsum(jnp.flip(sv))) + cs
        cnt_v[pl.ds(j * L, L)] = ksuf
        sum_v[pl.ds(j * L, L)] = ssuf
        idx = j * L + lax.iota(jnp.int32, L)
        edge = base + idx.astype(jnp.float32) * width
        f = ssuf - ksuf * edge - 1.0
        npos = npos + plsc.all_reduce_population_count(f > 0.0)
        return (ck + jnp.sum(kv), cs + jnp.sum(sv), npos)

    carry = (carry_k, carry_s, jnp.zeros((L,), jnp.int32))
    for i in range(NBLV):
        carry = body(i, carry)
    _, _, npos = carry
    jstar = jnp.maximum(jnp.max(npos) - 1, 0)
    jv = jnp.full((L,), jstar, jnp.int32)
    k_at = jnp.max(plsc.load_gather(cnt_v, [jv]))
    s_at = jnp.max(plsc.load_gather(sum_v, [jv]))
    jv1 = jnp.minimum(jv + 1, NBL - 1)
    top = jstar >= NBL - 1
    k_up = jnp.where(top, carry_k, jnp.max(plsc.load_gather(cnt_v, [jv1])))
    s_up = jnp.where(top, carry_s, jnp.max(plsc.load_gather(sum_v, [jv1])))
    return jstar, k_at, s_at, k_up, s_up


def _row_max(buf):
    def max_body(i, ms):
        return tuple(jnp.maximum(ms[j], buf[pl.ds((i + j) * L, L)])
                     for j in range(8))
    ms = plsc.parallel_loop(
        0, NV, 8, unroll=2,
        carry=tuple(jnp.full((L,), -jnp.inf, jnp.float32) for _ in range(8))
    )(max_body)
    m01 = jnp.maximum(jnp.maximum(ms[0], ms[1]), jnp.maximum(ms[2], ms[3]))
    m23 = jnp.maximum(jnp.maximum(ms[4], ms[5]), jnp.maximum(ms[6], ms[7]))
    return jnp.max(jnp.maximum(m01, m23))


def _row_tau(buf, lo, cmp_v, cmp2_v, cnt_v, sum_v):
    lanes = lax.iota(jnp.int32, L)
    ones = jnp.ones((L,), jnp.float32)

    def comp_body(i, cb):
        xv = buf[pl.ds(i * L, L)]
        p = plsc.all_reduce_population_count(xv > lo)
        any_v = p > 0
        plsc.store_scatter(cmp_v, [cb + lanes], xv, mask=any_v)
        return cb + jnp.where(any_v, L, 0)
    cb = plsc.parallel_loop(
        0, NV, unroll=8, carry=jnp.zeros((L,), jnp.int32))(comp_body)
    nc = jnp.max(cb)
    tc = (nc + (L - 1)) >> 4

    def comp2_body(i, cb2):
        for j in range(4):
            iv = i * 4 + j
            xv = cmp_v[pl.ds(iv * L, L)]
            msk = ((iv * L + lanes) < nc) & (xv > lo)
            mf = jnp.where(msk, jnp.float32(1.0), jnp.float32(0.0))
            pos = plsc.cumsum(mf).astype(jnp.int32)
            idx = jnp.minimum(jnp.maximum(cb2 + pos - 1, 0), CAP2 - 1)
            plsc.store_scatter(cmp2_v, [idx], xv, mask=msk)
            cb2 = cb2 + plsc.all_reduce_population_count(msk)
        return cb2
    cb2 = plsc.parallel_loop(
        0, (tc + 3) >> 2, carry=jnp.zeros((L,), jnp.int32))(comp2_body)
    nc2 = jnp.minimum(jnp.max(cb2), CAP2)
    tc2 = (nc2 + (L - 1)) >> 4

    base = lo
    scale = 1.0
    prev = []
    k_up = jnp.float32(0.0)
    s_up = jnp.float32(0.0)
    k_at = jnp.float32(0.0)
    s_at = jnp.float32(0.0)
    for _ in range(NLVL):
        scale = scale * NBL
        width = 1.0 / scale

        for i in range(NBLV):
            z = jnp.zeros((L,), jnp.float32)
            cnt_v[pl.ds(i * L, L)] = z
            sum_v[pl.ds(i * L, L)] = z

        def hist_body(i, _, base=base, scale=scale, prev=tuple(prev)):
            for j in range(4):
                iv = i * 4 + j
                xv = cmp2_v[pl.ds(iv * L, L)]
                msk = (iv * L + lanes) < nc2
                for (pb, ps, pj) in prev:
                    pbin = jnp.minimum(jnp.maximum(
                        ((xv - pb) * jnp.float32(ps)).astype(jnp.int32), 0),
                        NBL - 1)
                    msk = msk & (pbin == pj)
                b = jnp.minimum(jnp.maximum(
                    ((xv - base) * jnp.float32(scale)).astype(jnp.int32), 0),
                    NBL - 1)
                plsc.addupdate_scatter(cnt_v, [b], ones, mask=msk)
                plsc.addupdate_scatter(sum_v, [b], xv, mask=msk)
            return 0
        lax.fori_loop(0, (tc2 + 3) >> 2, hist_body, 0)

        jstar, k_at, s_at, k_up, s_up = _suffix_scan(
            cnt_v, sum_v, base, jnp.float32(width), k_up, s_up)
        prev.append((base, scale, jstar))
        base = base + jstar.astype(jnp.float32) * jnp.float32(width)

    num_v = jnp.full((L,), s_at - 1.0, jnp.float32)
    den_v = jnp.maximum(jnp.full((L,), k_at, jnp.float32), 1.0)
    return jnp.max(num_v / den_v)


def _sparsemax_body(x_hbm, out_hbm, buf0, buf1, cmp_v, cmp2_v, cnt_v, sum_v,
                    in_sems, out_sems):
    bufs = (buf0, buf1)
    wid = lax.axis_index("s") * NC + lax.axis_index("c")
    base_row = wid * ROWS_PER_W

    in_h = {0: pltpu.async_copy(x_hbm.at[base_row], bufs[0], in_sems.at[0])}
    out_h = {}
    for r in range(ROWS_PER_W):
        buf = bufs[r % 2]
        in_h[r].wait()
        lo = _row_max(buf) - 1.0
        if r + 1 < ROWS_PER_W:
            nxt = (r + 1) % 2
            if r >= 1:
                out_h[r - 1].wait()
            in_h[r + 1] = pltpu.async_copy(
                x_hbm.at[base_row + r + 1], bufs[nxt], in_sems.at[nxt])

        tau = _row_tau(buf, lo, cmp_v, cmp2_v, cnt_v, sum_v)

        def out_body(i):
            for j in range(8):
                xv = buf[pl.ds((i + j) * L, L)]
                buf[pl.ds((i + j) * L, L)] = jnp.maximum(xv - tau, 0.0)
        plsc.parallel_loop(0, NV, 8, unroll=2)(out_body)

        out_h[r] = pltpu.async_copy(
            buf, out_hbm.at[base_row + r], out_sems.at[r % 2])
    out_h[ROWS_PER_W - 2].wait()
    out_h[ROWS_PER_W - 1].wait()


def kernel(x):
    mesh = plsc.VectorSubcoreMesh(core_axis_name="c", subcore_axis_name="s")
    run = pl.kernel(
        _sparsemax_body,
        mesh=mesh,
        compiler_params=pltpu.CompilerParams(needs_layout_passes=False),
        out_type=jax.ShapeDtypeStruct((N_ROWS, N), jnp.float32),
        scratch_types=[
            pltpu.VMEM((N,), jnp.float32),
            pltpu.VMEM((N,), jnp.float32),
            pltpu.VMEM((N + 64,), jnp.float32),
            pltpu.VMEM((CAP2 + 64,), jnp.float32),
            pltpu.VMEM((NBL,), jnp.float32),
            pltpu.VMEM((NBL,), jnp.float32),
            pltpu.SemaphoreType.DMA((2,)),
            pltpu.SemaphoreType.DMA((2,)),
        ],
    )
    return run(x)
